# baseline (speedup 1.0000x reference)
import functools

import jax
import jax.numpy as jnp
from jax import lax
from jax.experimental import pallas as pl
from jax.experimental.pallas import tpu as pltpu
from jax.experimental.pallas import tpu_sc as plsc

_N = 10000
_E = 320000
_HID = 128
_LAT = 64

_NC = 2
_NS = 16
_NW = _NC * _NS

_RPT = 640
_NP = _NS * _RPT
_CHUNK = 128
_PW_CHUNKS = 80
_PW = _PW_CHUNKS * _CHUNK
_EP = _NW * _PW
_EROWS = _EP // _CHUNK

_mesh = plsc.VectorSubcoreMesh(
    core_axis_name="c", subcore_axis_name="s", num_cores=_NC, num_subcores=_NS
)


def _sc_deg_body(dst_hbm, zeros1_hbm, out_hbm, idx2_v, ones_v, acc_sh):
    c = lax.axis_index("c")
    s = lax.axis_index("s")
    w = s * _NC + c
    r0 = s * _RPT
    pltpu.sync_copy(zeros1_hbm.at[pl.ds(r0, _RPT)], acc_sh.at[pl.ds(r0, _RPT)])
    for i in range(_CHUNK // 16):
        ones_v[pl.ds(i * 16, 16)] = jnp.ones((16,), jnp.float32)
    pltpu.sync_copy(dst_hbm.at[pl.ds(w * _PW_CHUNKS, _PW_CHUNKS)], idx2_v)
    plsc.subcore_barrier()

    def body(k, carry):
        pltpu.sync_copy(ones_v, acc_sh.at[idx2_v.at[k]], add=True)
        return carry

    lax.fori_loop(0, _PW_CHUNKS, body, 0)
    plsc.subcore_barrier()
    pltpu.sync_copy(acc_sh.at[pl.ds(r0, _RPT)], out_hbm.at[c, pl.ds(r0, _RPT)])


_sc_deg = functools.partial(
    pl.kernel,
    out_type=jax.ShapeDtypeStruct((_NC, _NP), jnp.float32),
    mesh=_mesh,
    scratch_types=[
        pltpu.VMEM((_PW_CHUNKS, _CHUNK), jnp.int32),
        pltpu.VMEM((_CHUNK,), jnp.float32),
        pltpu.VMEM_SHARED((_NP,), jnp.float32),
    ],
)(_sc_deg_body)


def _sc_agg_body(table_hbm, src_hbm, dst_hbm, zeros2_hbm, out_hbm,
                 srcb_v, dstb_v, rows0_v, rows1_v, acc_sh,
                 gsem0, gsem1, ssem0, ssem1, lsem0, lsem1):
    c = lax.axis_index("c")
    s = lax.axis_index("s")
    w = s * _NC + c
    r0 = s * _RPT
    ebase = w * _PW_CHUNKS

    @pl.when(c == 0)
    def _():
        pltpu.sync_copy(table_hbm.at[pl.ds(r0, _RPT)], acc_sh.at[pl.ds(r0, _RPT)])

    @pl.when(c != 0)
    def _():
        pltpu.sync_copy(zeros2_hbm.at[pl.ds(r0, _RPT)], acc_sh.at[pl.ds(r0, _RPT)])

    rows = (rows0_v, rows1_v)
    gsems = (gsem0, gsem1)
    ssems = (ssem0, ssem1)
    lsems = (lsem0, lsem1)

    for b in (0, 1):
        pltpu.async_copy(src_hbm.at[pl.ds(ebase + b, 1)], srcb_v.at[pl.ds(b, 1)],
                         lsems[b])
        pltpu.async_copy(dst_hbm.at[pl.ds(ebase + b, 1)], dstb_v.at[pl.ds(b, 1)],
                         lsems[b])
    pltpu.make_async_copy(
        src_hbm.at[pl.ds(ebase, 1)], srcb_v.at[pl.ds(0, 1)], lsems[0]).wait()
    pltpu.make_async_copy(
        dst_hbm.at[pl.ds(ebase, 1)], dstb_v.at[pl.ds(0, 1)], lsems[0]).wait()
    pltpu.async_copy(table_hbm.at[srcb_v.at[0]], rows[0], gsems[0])
    plsc.subcore_barrier()

    def body(j2, carry):
        for p in (0, 1):
            for b in (0, 1):
                k = j2 * 4 + p * 2 + b
                slot = 2 * p + b
                nb = 1 - b
                nslot = (slot + 1) % 4

                pltpu.make_async_copy(
                    table_hbm.at[srcb_v.at[b]], rows[b], gsems[b]).wait()

                @pl.when((j2 > 0) | (p > 0) | (b > 0))
                def _():
                    pltpu.make_async_copy(
                        rows[nb], acc_sh.at[dstb_v.at[nslot]], ssems[nb]
                    ).wait()

                @pl.when(k + 1 < _PW_CHUNKS)
                def _():
                    pltpu.make_async_copy(
                        src_hbm.at[pl.ds(ebase, 1)], srcb_v.at[pl.ds(nb, 1)],
                        lsems[nb]).wait()
                    pltpu.make_async_copy(
                        dst_hbm.at[pl.ds(ebase, 1)],
                        dstb_v.at[pl.ds(nslot, 1)], lsems[nb]).wait()
                    pltpu.async_copy(table_hbm.at[srcb_v.at[nb]], rows[nb],
                                     gsems[nb])

                pltpu.async_copy(rows[b], acc_sh.at[dstb_v.at[slot]],
                                 ssems[b], add=True)

                @pl.when(k + 2 < _PW_CHUNKS)
                def _():
                    nrow = ebase + k + 2
                    pltpu.async_copy(src_hbm.at[pl.ds(nrow, 1)],
                                     srcb_v.at[pl.ds(b, 1)], lsems[b])
                    pltpu.async_copy(dst_hbm.at[pl.ds(nrow, 1)],
                                     dstb_v.at[pl.ds(2 * (1 - p) + b, 1)],
                                     lsems[b])
        return carry

    lax.fori_loop(0, _PW_CHUNKS // 4, body, 0)
    pltpu.make_async_copy(
        rows[1], acc_sh.at[dstb_v.at[3]], ssems[1]
    ).wait()
    plsc.subcore_barrier()
    pltpu.sync_copy(acc_sh.at[pl.ds(r0, _RPT)], out_hbm.at[c, pl.ds(r0, _RPT)])


_sc_agg = functools.partial(
    pl.kernel,
    out_type=jax.ShapeDtypeStruct((_NC, _NP, _HID), jnp.float32),
    mesh=_mesh,
    scratch_types=[
        pltpu.VMEM((2, _CHUNK), jnp.int32),
        pltpu.VMEM((4, _CHUNK), jnp.int32),
        pltpu.VMEM((_CHUNK, _HID), jnp.float32),
        pltpu.VMEM((_CHUNK, _HID), jnp.float32),
        pltpu.VMEM_SHARED((_NP, _HID), jnp.float32),
        pltpu.SemaphoreType.DMA,
        pltpu.SemaphoreType.DMA,
        pltpu.SemaphoreType.DMA,
        pltpu.SemaphoreType.DMA,
        pltpu.SemaphoreType.DMA,
        pltpu.SemaphoreType.DMA,
    ],
)(_sc_agg_body)


def _tc_matmul1_body(x_ref, w1_ref, y_ref):
    y_ref[...] = jnp.dot(x_ref[...], w1_ref[...],
                         preferred_element_type=jnp.float32)


def _tc_prep_body(y_ref, d0_ref, d1_ref, dinv_ref, ys_ref):
    deg = d0_ref[...] + d1_ref[...] + 1.0
    dinv = lax.rsqrt(deg)
    dinv_ref[...] = dinv
    ys_ref[...] = y_ref[...] * dinv


def _tc_layer1_body(p0_ref, p1_ref, dinv_ref, b1_ref, hs_ref):
    dinv = dinv_ref[...]
    h = (p0_ref[...] + p1_ref[...]) * dinv + b1_ref[...]
    h = jnp.where(h >= 0, h, 0.01 * h)
    hs_ref[...] = h * dinv


def _tc_layer2_body(q0_ref, q1_ref, dinv_ref, wmu_ref, bmu_ref, wlv_ref,
                    blv_ref, mu_ref, lv_ref):
    agg = ((q0_ref[...] + q1_ref[...]) * dinv_ref[...])[:_N]
    mu_ref[...] = (
        jnp.dot(agg, wmu_ref[...], preferred_element_type=jnp.float32)
        + bmu_ref[...]
    )
    lv = (
        jnp.dot(agg, wlv_ref[...], preferred_element_type=jnp.float32)
        + blv_ref[...]
    )
    lv_ref[...] = jnp.minimum(lv, 10.0)


def kernel(x, edge_index, W1, b1, Wmu, bmu, Wlv, blv):
    f32 = jnp.float32
    xp = jnp.zeros((_NP, _HID), f32).at[:_N].set(x)
    pad = _N + (jnp.arange(_EP - _E, dtype=jnp.int32) % (_NP - _N))
    srcp = jnp.concatenate([edge_index[0], pad]).reshape(_EROWS, _CHUNK)
    dstp = jnp.concatenate([edge_index[1], pad]).reshape(_EROWS, _CHUNK)
    zeros1 = jnp.zeros((_NP,), f32)
    zeros2 = jnp.zeros((_NP, _HID), f32)

    degp = _sc_deg(dstp, zeros1)
    d0 = degp[0].reshape(_NP, 1)
    d1 = degp[1].reshape(_NP, 1)

    y = pl.pallas_call(
        _tc_matmul1_body,
        out_shape=jax.ShapeDtypeStruct((_NP, _HID), f32),
    )(xp, W1)

    dinv, ys = pl.pallas_call(
        _tc_prep_body,
        out_shape=(
            jax.ShapeDtypeStruct((_NP, 1), f32),
            jax.ShapeDtypeStruct((_NP, _HID), f32),
        ),
    )(y, d0, d1)

    p = _sc_agg(ys, srcp, dstp, zeros2)

    hs = pl.pallas_call(
        _tc_layer1_body,
        out_shape=jax.ShapeDtypeStruct((_NP, _HID), f32),
    )(p[0], p[1], dinv, b1.reshape(1, _HID))

    q = _sc_agg(hs, srcp, dstp, zeros2)

    mu, logvar = pl.pallas_call(
        _tc_layer2_body,
        out_shape=(
            jax.ShapeDtypeStruct((_N, _LAT), f32),
            jax.ShapeDtypeStruct((_N, _LAT), f32),
        ),
    )(q[0], q[1], dinv, Wmu, bmu.reshape(1, _LAT), Wlv, blv.reshape(1, _LAT))
    return (mu, logvar)


# speedup vs baseline: 1.0730x; 1.0730x over previous
import functools

import jax
import jax.numpy as jnp
from jax import lax
from jax.experimental import pallas as pl
from jax.experimental.pallas import tpu as pltpu
from jax.experimental.pallas import tpu_sc as plsc

_N = 10000
_E = 320000
_HID = 128
_LAT = 64

_NC = 2
_NS = 16
_NW = _NC * _NS

_RPT = 640
_NP = _NS * _RPT
_CHUNK = 128
_PW_CHUNKS = 80
_PW = _PW_CHUNKS * _CHUNK
_EP = _NW * _PW
_EROWS = _EP // _CHUNK

_mesh = plsc.VectorSubcoreMesh(
    core_axis_name="c", subcore_axis_name="s", num_cores=_NC, num_subcores=_NS
)


def _sc_deg_body(dst_hbm, zeros1_hbm, out_hbm, idx2_v, ones_v, acc_sh):
    c = lax.axis_index("c")
    s = lax.axis_index("s")
    w = s * _NC + c
    r0 = s * _RPT
    pltpu.sync_copy(zeros1_hbm.at[pl.ds(r0, _RPT)], acc_sh.at[pl.ds(r0, _RPT)])
    for i in range(_CHUNK // 16):
        ones_v[pl.ds(i * 16, 16)] = jnp.ones((16,), jnp.float32)
    pltpu.sync_copy(dst_hbm.at[pl.ds(w * _PW_CHUNKS, _PW_CHUNKS)], idx2_v)
    plsc.subcore_barrier()

    def body(k, carry):
        pltpu.sync_copy(ones_v, acc_sh.at[idx2_v.at[k]], add=True)
        return carry

    lax.fori_loop(0, _PW_CHUNKS, body, 0)
    plsc.subcore_barrier()
    pltpu.sync_copy(acc_sh.at[pl.ds(r0, _RPT)], out_hbm.at[c, pl.ds(r0, _RPT)])


_sc_deg = functools.partial(
    pl.kernel,
    out_type=jax.ShapeDtypeStruct((_NC, _NP), jnp.float32),
    mesh=_mesh,
    scratch_types=[
        pltpu.VMEM((_PW_CHUNKS, _CHUNK), jnp.int32),
        pltpu.VMEM((_CHUNK,), jnp.float32),
        pltpu.VMEM_SHARED((_NP,), jnp.float32),
    ],
)(_sc_deg_body)


def _sc_agg_body(table_hbm, src_hbm, dst_hbm, zeros2_hbm, out_hbm,
                 srcb_v, dstb_v, rows0_v, rows1_v, acc_sh,
                 gsem0, gsem1, ssem0, ssem1, lsem0, lsem1):
    c = lax.axis_index("c")
    s = lax.axis_index("s")
    w = s * _NC + c
    r0 = s * _RPT
    ebase = w * _PW_CHUNKS

    @pl.when(c == 0)
    def _():
        pltpu.sync_copy(table_hbm.at[pl.ds(r0, _RPT)], acc_sh.at[pl.ds(r0, _RPT)])

    @pl.when(c != 0)
    def _():
        pltpu.sync_copy(zeros2_hbm.at[pl.ds(r0, _RPT)], acc_sh.at[pl.ds(r0, _RPT)])

    rows = (rows0_v, rows1_v)
    gsems = (gsem0, gsem1)
    ssems = (ssem0, ssem1)
    lsems = (lsem0, lsem1)

    for b in (0, 1):
        pltpu.async_copy(src_hbm.at[pl.ds(ebase + b, 1)], srcb_v.at[pl.ds(b, 1)],
                         lsems[b])
        pltpu.async_copy(dst_hbm.at[pl.ds(ebase + b, 1)], dstb_v.at[pl.ds(b, 1)],
                         lsems[b])
    pltpu.make_async_copy(
        src_hbm.at[pl.ds(ebase, 1)], srcb_v.at[pl.ds(0, 1)], lsems[0]).wait()
    pltpu.make_async_copy(
        dst_hbm.at[pl.ds(ebase, 1)], dstb_v.at[pl.ds(0, 1)], lsems[0]).wait()
    pltpu.async_copy(table_hbm.at[srcb_v.at[0]], rows[0], gsems[0])
    plsc.subcore_barrier()

    def body(j2, carry):
        for p in (0, 1):
            for b in (0, 1):
                k = j2 * 4 + p * 2 + b
                slot = 2 * p + b
                nb = 1 - b
                nslot = (slot + 1) % 4

                pltpu.make_async_copy(
                    table_hbm.at[srcb_v.at[b]], rows[b], gsems[b]).wait()

                @pl.when((j2 > 0) | (p > 0) | (b > 0))
                def _():
                    pltpu.make_async_copy(
                        rows[nb], acc_sh.at[dstb_v.at[nslot]], ssems[nb]
                    ).wait()

                @pl.when(k + 1 < _PW_CHUNKS)
                def _():
                    pltpu.make_async_copy(
                        src_hbm.at[pl.ds(ebase, 1)], srcb_v.at[pl.ds(nb, 1)],
                        lsems[nb]).wait()
                    pltpu.make_async_copy(
                        dst_hbm.at[pl.ds(ebase, 1)],
                        dstb_v.at[pl.ds(nslot, 1)], lsems[nb]).wait()
                    pltpu.async_copy(table_hbm.at[srcb_v.at[nb]], rows[nb],
                                     gsems[nb])

                pltpu.async_copy(rows[b], acc_sh.at[dstb_v.at[slot]],
                                 ssems[b], add=True)

                @pl.when(k + 2 < _PW_CHUNKS)
                def _():
                    nrow = ebase + k + 2
                    pltpu.async_copy(src_hbm.at[pl.ds(nrow, 1)],
                                     srcb_v.at[pl.ds(b, 1)], lsems[b])
                    pltpu.async_copy(dst_hbm.at[pl.ds(nrow, 1)],
                                     dstb_v.at[pl.ds(2 * (1 - p) + b, 1)],
                                     lsems[b])
        return carry

    lax.fori_loop(0, _PW_CHUNKS // 4, body, 0)
    pltpu.make_async_copy(
        rows[1], acc_sh.at[dstb_v.at[3]], ssems[1]
    ).wait()
    plsc.subcore_barrier()
    pltpu.sync_copy(acc_sh.at[pl.ds(r0, _RPT)], out_hbm.at[c, pl.ds(r0, _RPT)])


_sc_agg = functools.partial(
    pl.kernel,
    out_type=jax.ShapeDtypeStruct((_NC, _NP, _HID), jnp.float32),
    mesh=_mesh,
    scratch_types=[
        pltpu.VMEM((2, _CHUNK), jnp.int32),
        pltpu.VMEM((4, _CHUNK), jnp.int32),
        pltpu.VMEM((_CHUNK, _HID), jnp.float32),
        pltpu.VMEM((_CHUNK, _HID), jnp.float32),
        pltpu.VMEM_SHARED((_NP, _HID), jnp.float32),
        pltpu.SemaphoreType.DMA,
        pltpu.SemaphoreType.DMA,
        pltpu.SemaphoreType.DMA,
        pltpu.SemaphoreType.DMA,
        pltpu.SemaphoreType.DMA,
        pltpu.SemaphoreType.DMA,
    ],
)(_sc_agg_body)


def _tc_matmul1_body(x_ref, w1_ref, y_ref):
    y_ref[...] = jnp.dot(x_ref[...], w1_ref[...],
                         preferred_element_type=jnp.float32)


def _tc_prep_body(y_ref, deg_ref, dinv_ref, ys_ref):
    d = deg_ref[...]
    deg = (d[0:1, :] + d[1:2, :] + 1.0).reshape(_NP, 1)
    dinv = lax.rsqrt(deg)
    dinv_ref[...] = dinv
    ys_ref[...] = y_ref[...] * dinv


def _tc_layer1_body(p_ref, dinv_ref, b1_ref, hs_ref):
    dinv = dinv_ref[...]
    h = (p_ref[0] + p_ref[1]) * dinv + b1_ref[...]
    h = jnp.where(h >= 0, h, 0.01 * h)
    hs_ref[...] = h * dinv


def _tc_layer2_body(q_ref, dinv_ref, wmu_ref, bmu_ref, wlv_ref,
                    blv_ref, mu_ref, lv_ref):
    agg = ((q_ref[0] + q_ref[1]) * dinv_ref[...])[:_N]
    mu_ref[...] = (
        jnp.dot(agg, wmu_ref[...], preferred_element_type=jnp.float32)
        + bmu_ref[...]
    )
    lv = (
        jnp.dot(agg, wlv_ref[...], preferred_element_type=jnp.float32)
        + blv_ref[...]
    )
    lv_ref[...] = jnp.minimum(lv, 10.0)


def kernel(x, edge_index, W1, b1, Wmu, bmu, Wlv, blv):
    f32 = jnp.float32
    xp = jnp.zeros((_NP, _HID), f32).at[:_N].set(x)
    pad = _N + (jnp.arange(_EP - _E, dtype=jnp.int32) % (_NP - _N))
    srcp = jnp.concatenate([edge_index[0], pad]).reshape(_EROWS, _CHUNK)
    dstp = jnp.concatenate([edge_index[1], pad]).reshape(_EROWS, _CHUNK)
    zeros1 = jnp.zeros((_NP,), f32)
    zeros2 = jnp.zeros((_NP, _HID), f32)

    degp = _sc_deg(dstp, zeros1)

    y = pl.pallas_call(
        _tc_matmul1_body,
        out_shape=jax.ShapeDtypeStruct((_NP, _HID), f32),
    )(xp, W1)

    dinv, ys = pl.pallas_call(
        _tc_prep_body,
        out_shape=(
            jax.ShapeDtypeStruct((_NP, 1), f32),
            jax.ShapeDtypeStruct((_NP, _HID), f32),
        ),
    )(y, degp)

    p = _sc_agg(ys, srcp, dstp, zeros2)

    hs = pl.pallas_call(
        _tc_layer1_body,
        out_shape=jax.ShapeDtypeStruct((_NP, _HID), f32),
    )(p, dinv, b1.reshape(1, _HID))

    q = _sc_agg(hs, srcp, dstp, zeros2)

    mu, logvar = pl.pallas_call(
        _tc_layer2_body,
        out_shape=(
            jax.ShapeDtypeStruct((_N, _LAT), f32),
            jax.ShapeDtypeStruct((_N, _LAT), f32),
        ),
    )(q, dinv, Wmu, bmu.reshape(1, _LAT), Wlv, blv.reshape(1, _LAT))
    return (mu, logvar)
